# Initial kernel scaffold; baseline (speedup 1.0000x reference)
import jax
import jax.numpy as jnp
from jax.experimental import pallas as pl


def kernel(x, edge_index_list, edge_weights_list, W, b):
    raise NotImplementedError("write your pallas kernel here")


# speedup vs baseline: 2.1738x; 2.1738x over previous
import functools

import jax
import jax.numpy as jnp
from jax import lax
from jax.experimental import pallas as pl
from jax.experimental.pallas import tpu as pltpu
from jax.experimental.pallas import tpu_sc as plsc

NCONV = 3
KCH = 3
N = 10000
E = 320000
D = 128

NCORES = 2
NSUB = 16
NW = NCORES * NSUB
EW = E // NW
C = 64
EWP = 10240
NCH = EWP // C
W = 16
NWIN = NCH // W
NB = 2
RT = N // NSUB
RTA = 640
RTL = N - 15 * RTA
ZC = 40
DV = D // 16


def _sc_mesh():
    return plsc.VectorSubcoreMesh(core_axis_name="c", subcore_axis_name="s",
                                  num_cores=NCORES, num_subcores=NSUB)


def _deg_body(src_hbm, ew_hbm, rid_hbm, deg_out,
              acc_deg, src_v, ew_v, rid_v, deg_l, deg_l2):
    cid = lax.axis_index("c")
    sid = lax.axis_index("s")
    wid = cid * NSUB + sid
    zeros16 = jnp.zeros((16,), jnp.float32)

    pltpu.sync_copy(rid_hbm, rid_v)

    @pl.loop(0, NCONV)
    def _(i):
        @pl.loop(0, RT)
        def _(j):
            deg_l[pl.ds(j * 16, 16)] = zeros16
            deg_l2[j] = zeros16

        @pl.when(sid == 0)
        def _():
            pltpu.sync_copy(deg_l2, acc_deg)

        pltpu.sync_copy(src_hbm.at[i, wid], src_v)
        pltpu.sync_copy(ew_hbm.at[i, wid], ew_v)

        plsc.subcore_barrier()

        @pl.loop(0, NCH)
        def _(j):
            for k in range(C // 16):
                sl = pl.ds(k * 16, 16)
                plsc.addupdate_scatter(deg_l, [src_v[j, sl]], ew_v[j, sl])

        @pl.loop(0, RT)
        def _(j):
            deg_l2[j] = deg_l[pl.ds(j * 16, 16)]

        for k in range(5):
            pltpu.sync_copy(deg_l2.at[pl.ds(k * 125, 125)],
                            acc_deg.at[rid_v.at[k]], add=True)

        plsc.subcore_barrier()

        @pl.when(sid == 0)
        def _():
            pltpu.sync_copy(acc_deg, deg_out.at[cid, i])


@functools.lru_cache(maxsize=None)
def _make_deg_pass():
    return pl.kernel(
        _deg_body,
        out_type=(jax.ShapeDtypeStruct((NCORES, NCONV, RT, 16), jnp.float32),),
        mesh=_sc_mesh(),
        scratch_types=[
            pltpu.VMEM_SHARED((RT, 16), jnp.float32),
            pltpu.VMEM((NCH, C), jnp.int32),
            pltpu.VMEM((NCH, C), jnp.float32),
            pltpu.VMEM((5, 125), jnp.int32),
            pltpu.VMEM((N,), jnp.float32),
            pltpu.VMEM((RT, 16), jnp.float32),
        ],
        compiler_params=pltpu.CompilerParams(needs_layout_passes=False, use_tc_tiling_on_sc=False),
    )


def _edge_body(table, src_hbm, dst_hbm, ew_hbm, agg_out,
               acc, src_w, dst_w, ew_w, rows, zbuf, gsem, ssem):
    cid = lax.axis_index("c")
    sid = lax.axis_index("s")
    wid = cid * NSUB + sid

    _zero = jnp.zeros((16,), jnp.float32)

    @pl.loop(0, ZC)
    def _(j):
        for v in range(DV):
            zbuf[j, pl.ds(v * 16, 16)] = _zero

    @pl.loop(0, NCONV)
    def _(i):
        @pl.when(sid < 15)
        def _():
            @pl.loop(0, RTA // ZC)
            def _(z):
                pltpu.sync_copy(zbuf, acc.at[pl.ds(sid * RTA + z * ZC, ZC)])

        @pl.when(sid == 15)
        def _():
            @pl.loop(0, RTL // ZC)
            def _(z):
                pltpu.sync_copy(zbuf, acc.at[pl.ds(15 * RTA + z * ZC, ZC)])

        plsc.subcore_barrier()

        @pl.loop(0, NWIN)
        def _(w):
            pltpu.sync_copy(src_hbm.at[i, wid, pl.ds(w * W, W)], src_w)
            pltpu.sync_copy(dst_hbm.at[i, wid, pl.ds(w * W, W)], dst_w)
            pltpu.sync_copy(ew_hbm.at[i, wid, pl.ds(w * W, W)], ew_w)

            @pl.loop(0, W, step=NB)
            def _(j0):
                gds = []
                for bb in range(NB):
                    gds.append(pltpu.async_copy(
                        table.at[src_w.at[j0 + bb]],
                        rows.at[pl.ds(bb * C, C)], gsem))
                for dsc in gds:
                    dsc.wait()
                for bb in range(NB):
                    j = j0 + bb

                    @pl.loop(0, C // 16)
                    def _(k):
                        wneg = ew_w[j, pl.ds(k * 16, 16)] * -1.0
                        for t in range(16):
                            r = bb * C + k * 16 + t
                            vw = jnp.broadcast_to(wneg[t], (16,))
                            for v in range(DV):
                                sl = pl.ds(v * 16, 16)
                                rows[r, sl] = rows[r, sl] * vw
                sds = []
                for bb in range(NB):
                    sds.append(pltpu.async_copy(
                        rows.at[pl.ds(bb * C, C)],
                        acc.at[dst_w.at[j0 + bb]], ssem, add=True))
                for dsc in sds:
                    dsc.wait()

        plsc.subcore_barrier()

        @pl.when(sid < 15)
        def _():
            pltpu.sync_copy(acc.at[pl.ds(sid * RTA, RTA)],
                            agg_out.at[cid, i, pl.ds(sid * RTA, RTA)])

        @pl.when(sid == 15)
        def _():
            pltpu.sync_copy(acc.at[pl.ds(15 * RTA, RTL)],
                            agg_out.at[cid, i, pl.ds(15 * RTA, RTL)])


@functools.lru_cache(maxsize=None)
def _make_edge_pass(table_rows):
    def body(table, src, dst, ew, agg_out, *scr):
        _edge_body(table, src, dst, ew, agg_out, *scr)

    return pl.kernel(
        body,
        out_type=(jax.ShapeDtypeStruct((NCORES, NCONV, N, D), jnp.float32),),
        mesh=_sc_mesh(),
        scratch_types=[
            pltpu.VMEM_SHARED((N, D), jnp.float32),
            pltpu.VMEM((W, C), jnp.int32),
            pltpu.VMEM((W, C), jnp.int32),
            pltpu.VMEM((W, C), jnp.float32),
            pltpu.VMEM((NB * C, D), jnp.float32),
            pltpu.VMEM((ZC, D), jnp.float32),
            pltpu.SemaphoreType.DMA,
            pltpu.SemaphoreType.DMA,
        ],
        compiler_params=pltpu.CompilerParams(needs_layout_passes=False, use_tc_tiling_on_sc=False),
    )


BN = 1000


def _tc_a_body(agg_ref, deg_ref, x_ref, tx1_ref, wdiag_ref):
    wd = deg_ref[0, 0] + deg_ref[1, 0] - 1.0
    wdiag_ref[0] = wd
    tx1_ref[0] = agg_ref[0, 0] + agg_ref[1, 0] + wd * x_ref[...]


def _tc_b_body(x_ref, tx1_ref, agg2_ref, wdiag_ref, w_ref, b_ref, out_ref):
    xb = x_ref[...]
    acc = b_ref[0:1] + b_ref[1:2] + b_ref[2:3]
    acc = jnp.broadcast_to(acc, (BN, D)).astype(jnp.float32)
    for i in range(NCONV):
        t1 = tx1_ref[i]
        t2 = 2.0 * (agg2_ref[0, i] + agg2_ref[1, i] + wdiag_ref[i] * t1) - xb
        acc = acc + jnp.dot(xb, w_ref[i, 0], precision=lax.Precision.HIGHEST,
                            preferred_element_type=jnp.float32)
        acc = acc + jnp.dot(t1, w_ref[i, 1], precision=lax.Precision.HIGHEST,
                            preferred_element_type=jnp.float32)
        acc = acc + jnp.dot(t2, w_ref[i, 2], precision=lax.Precision.HIGHEST,
                            preferred_element_type=jnp.float32)
    out_ref[...] = acc


def _pad_edges(a, fill):
    pad = jnp.full((NCONV, NW, EWP - EW), fill, a.dtype)
    return jnp.concatenate([a.reshape(NCONV, NW, EW), pad], axis=2).reshape(
        NCONV, NW, NCH, C)


@jax.jit
def kernel(x, edge_index_list, edge_weights_list, W_, b):
    src = _pad_edges(edge_index_list[:, 0], 0)
    dst = _pad_edges(edge_index_list[:, 1], 0)
    ew = _pad_edges(edge_weights_list, 0.0)
    src2 = src + (jnp.arange(NCONV, dtype=jnp.int32) * N)[:, None, None, None]
    rid = jnp.arange(RT, dtype=jnp.int32).reshape(5, 125)

    (deg,) = _make_deg_pass()(src, ew, rid)
    deg4 = deg.reshape(NCORES, NCONV, N, 1)

    (agg1,) = _make_edge_pass(N)(x, src, dst, ew)

    tx1, wdiag = pl.pallas_call(
        _tc_a_body,
        grid=(NCONV, N // BN),
        in_specs=[
            pl.BlockSpec((NCORES, 1, BN, D), lambda i, r: (0, i, r, 0)),
            pl.BlockSpec((NCORES, 1, BN, 1), lambda i, r: (0, i, r, 0)),
            pl.BlockSpec((BN, D), lambda i, r: (r, 0)),
        ],
        out_specs=[
            pl.BlockSpec((1, BN, D), lambda i, r: (i, r, 0)),
            pl.BlockSpec((1, BN, 1), lambda i, r: (i, r, 0)),
        ],
        out_shape=[
            jax.ShapeDtypeStruct((NCONV, N, D), jnp.float32),
            jax.ShapeDtypeStruct((NCONV, N, 1), jnp.float32),
        ],
    )(agg1, deg4, x)

    (agg2,) = _make_edge_pass(NCONV * N)(
        tx1.reshape(NCONV * N, D), src2, dst, ew)

    out = pl.pallas_call(
        _tc_b_body,
        grid=(N // BN,),
        in_specs=[
            pl.BlockSpec((BN, D), lambda r: (r, 0)),
            pl.BlockSpec((NCONV, BN, D), lambda r: (0, r, 0)),
            pl.BlockSpec((NCORES, NCONV, BN, D), lambda r: (0, 0, r, 0)),
            pl.BlockSpec((NCONV, BN, 1), lambda r: (0, r, 0)),
            pl.BlockSpec((NCONV, KCH, D, D), lambda r: (0, 0, 0, 0)),
            pl.BlockSpec((NCONV, D), lambda r: (0, 0)),
        ],
        out_specs=pl.BlockSpec((BN, D), lambda r: (r, 0)),
        out_shape=jax.ShapeDtypeStruct((N, D), jnp.float32),
    )(x, tx1, agg2, wdiag, W_, b)
    return out


# speedup vs baseline: 2.3953x; 1.1019x over previous
import functools

import jax
import jax.numpy as jnp
from jax import lax
from jax.experimental import pallas as pl
from jax.experimental.pallas import tpu as pltpu
from jax.experimental.pallas import tpu_sc as plsc

NCONV = 3
KCH = 3
N = 10000
E = 320000
D = 128

NCORES = 2
NSUB = 16
NW = NCORES * NSUB
EW = E // NW
C = 64
EWP = 10240
NCH = EWP // C
W = 16
NWIN = NCH // W
NB = 4
RT = N // NSUB
RTA = 640
RTL = N - 15 * RTA
ZC = 40
DV = D // 16


def _sc_mesh():
    return plsc.VectorSubcoreMesh(core_axis_name="c", subcore_axis_name="s",
                                  num_cores=NCORES, num_subcores=NSUB)


def _deg_body(src_hbm, ew_hbm, rid_hbm, deg_out,
              acc_deg, src_v, ew_v, rid_v, deg_l, deg_l2):
    cid = lax.axis_index("c")
    sid = lax.axis_index("s")
    wid = cid * NSUB + sid
    zeros16 = jnp.zeros((16,), jnp.float32)

    pltpu.sync_copy(rid_hbm, rid_v)

    @pl.loop(0, NCONV)
    def _(i):
        @pl.loop(0, RT)
        def _(j):
            deg_l[pl.ds(j * 16, 16)] = zeros16
            deg_l2[j] = zeros16

        @pl.when(sid == 0)
        def _():
            pltpu.sync_copy(deg_l2, acc_deg)

        pltpu.sync_copy(src_hbm.at[i, wid], src_v)
        pltpu.sync_copy(ew_hbm.at[i, wid], ew_v)

        plsc.subcore_barrier()

        @pl.loop(0, NCH)
        def _(j):
            for k in range(C // 16):
                sl = pl.ds(k * 16, 16)
                plsc.addupdate_scatter(deg_l, [src_v[j, sl]], ew_v[j, sl])

        @pl.loop(0, RT)
        def _(j):
            deg_l2[j] = deg_l[pl.ds(j * 16, 16)]

        for k in range(5):
            pltpu.sync_copy(deg_l2.at[pl.ds(k * 125, 125)],
                            acc_deg.at[rid_v.at[k]], add=True)

        plsc.subcore_barrier()

        @pl.when(sid == 0)
        def _():
            pltpu.sync_copy(acc_deg, deg_out.at[cid, i])


@functools.lru_cache(maxsize=None)
def _make_deg_pass():
    return pl.kernel(
        _deg_body,
        out_type=(jax.ShapeDtypeStruct((NCORES, NCONV, RT, 16), jnp.float32),),
        mesh=_sc_mesh(),
        scratch_types=[
            pltpu.VMEM_SHARED((RT, 16), jnp.float32),
            pltpu.VMEM((NCH, C), jnp.int32),
            pltpu.VMEM((NCH, C), jnp.float32),
            pltpu.VMEM((5, 125), jnp.int32),
            pltpu.VMEM((N,), jnp.float32),
            pltpu.VMEM((RT, 16), jnp.float32),
        ],
        compiler_params=pltpu.CompilerParams(needs_layout_passes=False, use_tc_tiling_on_sc=False),
    )


def _edge_body(table, src_hbm, dst_hbm, ew_hbm, agg_out,
               acc, src_w, dst_w, ew_w, rows, zbuf, gsem, ssem):
    cid = lax.axis_index("c")
    sid = lax.axis_index("s")
    wid = cid * NSUB + sid

    _zero = jnp.zeros((16,), jnp.float32)

    @pl.loop(0, ZC)
    def _(j):
        for v in range(DV):
            zbuf[j, pl.ds(v * 16, 16)] = _zero

    @pl.loop(0, NCONV)
    def _(i):
        @pl.when(sid < 15)
        def _():
            @pl.loop(0, RTA // ZC)
            def _(z):
                pltpu.sync_copy(zbuf, acc.at[pl.ds(sid * RTA + z * ZC, ZC)])

        @pl.when(sid == 15)
        def _():
            @pl.loop(0, RTL // ZC)
            def _(z):
                pltpu.sync_copy(zbuf, acc.at[pl.ds(15 * RTA + z * ZC, ZC)])

        plsc.subcore_barrier()

        @pl.loop(0, NWIN)
        def _(w):
            pltpu.sync_copy(src_hbm.at[i, wid, pl.ds(w * W, W)], src_w)
            pltpu.sync_copy(dst_hbm.at[i, wid, pl.ds(w * W, W)], dst_w)
            pltpu.sync_copy(ew_hbm.at[i, wid, pl.ds(w * W, W)], ew_w)

            @pl.loop(0, W, step=NB)
            def _(j0):
                gds = []
                for bb in range(NB):
                    gds.append(pltpu.async_copy(
                        table.at[src_w.at[j0 + bb]],
                        rows.at[pl.ds(bb * C, C)], gsem.at[bb]))
                sds = []
                for bb in range(NB):
                    j = j0 + bb
                    gds[bb].wait()

                    @pl.loop(0, C // 16)
                    def _(k):
                        wneg = ew_w[j, pl.ds(k * 16, 16)] * -1.0
                        for t in range(16):
                            r = bb * C + k * 16 + t
                            vw = jnp.broadcast_to(wneg[t], (16,))
                            for v in range(DV):
                                sl = pl.ds(v * 16, 16)
                                rows[r, sl] = rows[r, sl] * vw
                    sds.append(pltpu.async_copy(
                        rows.at[pl.ds(bb * C, C)],
                        acc.at[dst_w.at[j]], ssem.at[bb], add=True))
                for dsc in sds:
                    dsc.wait()

        plsc.subcore_barrier()

        @pl.when(sid < 15)
        def _():
            pltpu.sync_copy(acc.at[pl.ds(sid * RTA, RTA)],
                            agg_out.at[cid, i, pl.ds(sid * RTA, RTA)])

        @pl.when(sid == 15)
        def _():
            pltpu.sync_copy(acc.at[pl.ds(15 * RTA, RTL)],
                            agg_out.at[cid, i, pl.ds(15 * RTA, RTL)])


@functools.lru_cache(maxsize=None)
def _make_edge_pass(table_rows):
    def body(table, src, dst, ew, agg_out, *scr):
        _edge_body(table, src, dst, ew, agg_out, *scr)

    return pl.kernel(
        body,
        out_type=(jax.ShapeDtypeStruct((NCORES, NCONV, N, D), jnp.float32),),
        mesh=_sc_mesh(),
        scratch_types=[
            pltpu.VMEM_SHARED((N, D), jnp.float32),
            pltpu.VMEM((W, C), jnp.int32),
            pltpu.VMEM((W, C), jnp.int32),
            pltpu.VMEM((W, C), jnp.float32),
            pltpu.VMEM((NB * C, D), jnp.float32),
            pltpu.VMEM((ZC, D), jnp.float32),
            pltpu.SemaphoreType.DMA((NB,)),
            pltpu.SemaphoreType.DMA((NB,)),
        ],
        compiler_params=pltpu.CompilerParams(needs_layout_passes=False, use_tc_tiling_on_sc=False),
    )


BN = 1000


def _tc_a_body(agg_ref, deg_ref, x_ref, tx1_ref, wdiag_ref):
    wd = deg_ref[0, 0] + deg_ref[1, 0] - 1.0
    wdiag_ref[0] = wd
    tx1_ref[0] = agg_ref[0, 0] + agg_ref[1, 0] + wd * x_ref[...]


def _tc_b_body(x_ref, tx1_ref, agg2_ref, wdiag_ref, w_ref, b_ref, out_ref):
    xb = x_ref[...]
    acc = b_ref[0:1] + b_ref[1:2] + b_ref[2:3]
    acc = jnp.broadcast_to(acc, (BN, D)).astype(jnp.float32)
    for i in range(NCONV):
        t1 = tx1_ref[i]
        t2 = 2.0 * (agg2_ref[0, i] + agg2_ref[1, i] + wdiag_ref[i] * t1) - xb
        acc = acc + jnp.dot(xb, w_ref[i, 0], precision=lax.Precision.HIGHEST,
                            preferred_element_type=jnp.float32)
        acc = acc + jnp.dot(t1, w_ref[i, 1], precision=lax.Precision.HIGHEST,
                            preferred_element_type=jnp.float32)
        acc = acc + jnp.dot(t2, w_ref[i, 2], precision=lax.Precision.HIGHEST,
                            preferred_element_type=jnp.float32)
    out_ref[...] = acc


def _pad_edges(a, fill):
    pad = jnp.full((NCONV, NW, EWP - EW), fill, a.dtype)
    return jnp.concatenate([a.reshape(NCONV, NW, EW), pad], axis=2).reshape(
        NCONV, NW, NCH, C)


@jax.jit
def kernel(x, edge_index_list, edge_weights_list, W_, b):
    src = _pad_edges(edge_index_list[:, 0], 0)
    dst = _pad_edges(edge_index_list[:, 1], 0)
    ew = _pad_edges(edge_weights_list, 0.0)
    src2 = src + (jnp.arange(NCONV, dtype=jnp.int32) * N)[:, None, None, None]
    rid = jnp.arange(RT, dtype=jnp.int32).reshape(5, 125)

    (deg,) = _make_deg_pass()(src, ew, rid)
    deg4 = deg.reshape(NCORES, NCONV, N, 1)

    (agg1,) = _make_edge_pass(N)(x, src, dst, ew)

    tx1, wdiag = pl.pallas_call(
        _tc_a_body,
        grid=(NCONV, N // BN),
        in_specs=[
            pl.BlockSpec((NCORES, 1, BN, D), lambda i, r: (0, i, r, 0)),
            pl.BlockSpec((NCORES, 1, BN, 1), lambda i, r: (0, i, r, 0)),
            pl.BlockSpec((BN, D), lambda i, r: (r, 0)),
        ],
        out_specs=[
            pl.BlockSpec((1, BN, D), lambda i, r: (i, r, 0)),
            pl.BlockSpec((1, BN, 1), lambda i, r: (i, r, 0)),
        ],
        out_shape=[
            jax.ShapeDtypeStruct((NCONV, N, D), jnp.float32),
            jax.ShapeDtypeStruct((NCONV, N, 1), jnp.float32),
        ],
    )(agg1, deg4, x)

    (agg2,) = _make_edge_pass(NCONV * N)(
        tx1.reshape(NCONV * N, D), src2, dst, ew)

    out = pl.pallas_call(
        _tc_b_body,
        grid=(N // BN,),
        in_specs=[
            pl.BlockSpec((BN, D), lambda r: (r, 0)),
            pl.BlockSpec((NCONV, BN, D), lambda r: (0, r, 0)),
            pl.BlockSpec((NCORES, NCONV, BN, D), lambda r: (0, 0, r, 0)),
            pl.BlockSpec((NCONV, BN, 1), lambda r: (0, r, 0)),
            pl.BlockSpec((NCONV, KCH, D, D), lambda r: (0, 0, 0, 0)),
            pl.BlockSpec((NCONV, D), lambda r: (0, 0)),
        ],
        out_specs=pl.BlockSpec((BN, D), lambda r: (r, 0)),
        out_shape=jax.ShapeDtypeStruct((N, D), jnp.float32),
    )(x, tx1, agg2, wdiag, W_, b)
    return out


# speedup vs baseline: 2.4849x; 1.0374x over previous
import functools

import jax
import jax.numpy as jnp
from jax import lax
from jax.experimental import pallas as pl
from jax.experimental.pallas import tpu as pltpu
from jax.experimental.pallas import tpu_sc as plsc

NCONV = 3
KCH = 3
N = 10000
E = 320000
D = 128
DH = D // 2

NCORES = 2
NSUB = 16
NW = NCORES * NSUB
EW = E // NW
C = 64
EWP = 10240
NCH = EWP // C
W = 80
NWIN = NCH // W
NB = 4
RT = N // NSUB
RTA = 640
RTL = N - 15 * RTA
ZC = 40
DV = DH // 16


def _sc_mesh():
    return plsc.VectorSubcoreMesh(core_axis_name="c", subcore_axis_name="s",
                                  num_cores=NCORES, num_subcores=NSUB)


def _deg_body(src_hbm, ew_hbm, rid_hbm, deg_out,
              acc_deg, src_v, ew_v, rid_v, deg_l, deg_l2):
    cid = lax.axis_index("c")
    sid = lax.axis_index("s")
    wid = cid * NSUB + sid
    zeros16 = jnp.zeros((16,), jnp.float32)

    pltpu.sync_copy(rid_hbm, rid_v)

    @pl.loop(0, NCONV)
    def _(i):
        @pl.loop(0, RT)
        def _(j):
            deg_l[pl.ds(j * 16, 16)] = zeros16
            deg_l2[j] = zeros16

        @pl.when(sid == 0)
        def _():
            pltpu.sync_copy(deg_l2, acc_deg)

        pltpu.sync_copy(src_hbm.at[i, wid], src_v)
        pltpu.sync_copy(ew_hbm.at[i, wid], ew_v)

        plsc.subcore_barrier()

        @pl.loop(0, NCH)
        def _(j):
            for k in range(C // 16):
                sl = pl.ds(k * 16, 16)
                plsc.addupdate_scatter(deg_l, [src_v[j, sl]], ew_v[j, sl])

        @pl.loop(0, RT)
        def _(j):
            deg_l2[j] = deg_l[pl.ds(j * 16, 16)]

        for k in range(5):
            pltpu.sync_copy(deg_l2.at[pl.ds(k * 125, 125)],
                            acc_deg.at[rid_v.at[k]], add=True)

        plsc.subcore_barrier()

        @pl.when(sid == 0)
        def _():
            pltpu.sync_copy(acc_deg, deg_out.at[cid, i])


@functools.lru_cache(maxsize=None)
def _make_deg_pass():
    return pl.kernel(
        _deg_body,
        out_type=(jax.ShapeDtypeStruct((NCORES, NCONV, RT, 16), jnp.float32),),
        mesh=_sc_mesh(),
        scratch_types=[
            pltpu.VMEM_SHARED((RT, 16), jnp.float32),
            pltpu.VMEM((NCH, C), jnp.int32),
            pltpu.VMEM((NCH, C), jnp.float32),
            pltpu.VMEM((5, 125), jnp.int32),
            pltpu.VMEM((N,), jnp.float32),
            pltpu.VMEM((RT, 16), jnp.float32),
        ],
        compiler_params=pltpu.CompilerParams(
            needs_layout_passes=False, use_tc_tiling_on_sc=False),
    )


def _edge_body(nconv_tables, tables, src_hbm, dst_hbm, ew_hbm, agg_out,
               xs, acc, src_w, dst_w, ew_w, rows, zbuf, gsem, ssem):
    cid = lax.axis_index("c")
    sid = lax.axis_index("s")
    wid = cid * NSUB + sid

    _zero = jnp.zeros((16,), jnp.float32)

    @pl.loop(0, ZC)
    def _(j):
        for v in range(DV):
            zbuf[j, pl.ds(v * 16, 16)] = _zero

    @pl.loop(0, 2)
    def _(h):
        @pl.loop(0, NCONV)
        def _(i):
            ti = i if nconv_tables > 1 else 0

            @pl.when(sid < 15)
            def _():
                pltpu.sync_copy(tables.at[ti, h, pl.ds(sid * RTA, RTA)],
                                xs.at[pl.ds(sid * RTA, RTA)])

                @pl.loop(0, RTA // ZC)
                def _(z):
                    pltpu.sync_copy(zbuf,
                                    acc.at[pl.ds(sid * RTA + z * ZC, ZC)])

            @pl.when(sid == 15)
            def _():
                pltpu.sync_copy(tables.at[ti, h, pl.ds(15 * RTA, RTL)],
                                xs.at[pl.ds(15 * RTA, RTL)])

                @pl.loop(0, RTL // ZC)
                def _(z):
                    pltpu.sync_copy(zbuf,
                                    acc.at[pl.ds(15 * RTA + z * ZC, ZC)])

            plsc.subcore_barrier()

            @pl.loop(0, NWIN)
            def _(w):
                pltpu.sync_copy(src_hbm.at[i, wid, pl.ds(w * W, W)], src_w)
                pltpu.sync_copy(dst_hbm.at[i, wid, pl.ds(w * W, W)], dst_w)
                pltpu.sync_copy(ew_hbm.at[i, wid, pl.ds(w * W, W)], ew_w)

                @pl.loop(0, W, step=NB)
                def _(j0):
                    gds = []
                    for bb in range(NB):
                        gds.append(pltpu.async_copy(
                            xs.at[src_w.at[j0 + bb]],
                            rows.at[pl.ds(bb * C, C)], gsem.at[bb]))
                    sds = []
                    for bb in range(NB):
                        j = j0 + bb
                        gds[bb].wait()

                        @pl.loop(0, C // 16)
                        def _(k):
                            wneg = ew_w[j, pl.ds(k * 16, 16)] * -1.0
                            for t in range(16):
                                r = bb * C + k * 16 + t
                                vw = jnp.broadcast_to(wneg[t], (16,))
                                for v in range(DV):
                                    sl = pl.ds(v * 16, 16)
                                    rows[r, sl] = rows[r, sl] * vw
                        sds.append(pltpu.async_copy(
                            rows.at[pl.ds(bb * C, C)],
                            acc.at[dst_w.at[j]], ssem.at[bb], add=True))
                    for dsc in sds:
                        dsc.wait()

            plsc.subcore_barrier()

            @pl.when(sid < 15)
            def _():
                pltpu.sync_copy(acc.at[pl.ds(sid * RTA, RTA)],
                                agg_out.at[cid, i, h, pl.ds(sid * RTA, RTA)])

            @pl.when(sid == 15)
            def _():
                pltpu.sync_copy(acc.at[pl.ds(15 * RTA, RTL)],
                                agg_out.at[cid, i, h, pl.ds(15 * RTA, RTL)])


@functools.lru_cache(maxsize=None)
def _make_edge_pass(nconv_tables):
    body = functools.partial(_edge_body, nconv_tables)

    return pl.kernel(
        body,
        out_type=(jax.ShapeDtypeStruct((NCORES, NCONV, 2, N, DH),
                                       jnp.float32),),
        mesh=_sc_mesh(),
        scratch_types=[
            pltpu.VMEM_SHARED((N, DH), jnp.float32),
            pltpu.VMEM_SHARED((N, DH), jnp.float32),
            pltpu.VMEM((W, C), jnp.int32),
            pltpu.VMEM((W, C), jnp.int32),
            pltpu.VMEM((W, C), jnp.float32),
            pltpu.VMEM((NB * C, DH), jnp.float32),
            pltpu.VMEM((ZC, DH), jnp.float32),
            pltpu.SemaphoreType.DMA((NB,)),
            pltpu.SemaphoreType.DMA((NB,)),
        ],
        compiler_params=pltpu.CompilerParams(
            needs_layout_passes=False, use_tc_tiling_on_sc=False),
    )


BN = 1000


def _tc_a_body(agg_ref, deg_ref, xh_ref, tx1h_ref, wdiag_ref):
    wd = deg_ref[0, 0] + deg_ref[1, 0] - 1.0
    wdiag_ref[0] = wd
    tx1h_ref[0, 0] = (agg_ref[0, 0, 0] + agg_ref[1, 0, 0]
                      + wd * xh_ref[0])


def _tc_b_body(x_ref, tx1h_ref, agg2_ref, wdiag_ref, w_ref, b_ref, out_ref):
    xb = x_ref[...]
    acc = b_ref[0:1] + b_ref[1:2] + b_ref[2:3]
    acc = jnp.broadcast_to(acc, (BN, D)).astype(jnp.float32)
    for i in range(NCONV):
        t1 = jnp.concatenate([tx1h_ref[i, 0], tx1h_ref[i, 1]], axis=-1)
        a2 = jnp.concatenate(
            [agg2_ref[0, i, 0] + agg2_ref[1, i, 0],
             agg2_ref[0, i, 1] + agg2_ref[1, i, 1]], axis=-1)
        t2 = 2.0 * (a2 + wdiag_ref[i] * t1) - xb
        acc = acc + jnp.dot(xb, w_ref[i, 0], precision=lax.Precision.HIGHEST,
                            preferred_element_type=jnp.float32)
        acc = acc + jnp.dot(t1, w_ref[i, 1], precision=lax.Precision.HIGHEST,
                            preferred_element_type=jnp.float32)
        acc = acc + jnp.dot(t2, w_ref[i, 2], precision=lax.Precision.HIGHEST,
                            preferred_element_type=jnp.float32)
    out_ref[...] = acc


def _pad_edges(a, fill):
    pad = jnp.full((NCONV, NW, EWP - EW), fill, a.dtype)
    return jnp.concatenate([a.reshape(NCONV, NW, EW), pad], axis=2).reshape(
        NCONV, NW, NCH, C)


@jax.jit
def kernel(x, edge_index_list, edge_weights_list, W_, b):
    src = _pad_edges(edge_index_list[:, 0], 0)
    dst = _pad_edges(edge_index_list[:, 1], 0)
    ew = _pad_edges(edge_weights_list, 0.0)
    rid = jnp.arange(RT, dtype=jnp.int32).reshape(5, 125)
    xh = jnp.stack([x[:, :DH], x[:, DH:]])[None]

    (deg,) = _make_deg_pass()(src, ew, rid)
    deg4 = deg.reshape(NCORES, NCONV, N, 1)

    (agg1,) = _make_edge_pass(1)(xh, src, dst, ew)

    tx1h, wdiag = pl.pallas_call(
        _tc_a_body,
        grid=(NCONV, 2, N // BN),
        in_specs=[
            pl.BlockSpec((NCORES, 1, 1, BN, DH),
                         lambda i, h, r: (0, i, h, r, 0)),
            pl.BlockSpec((NCORES, 1, BN, 1), lambda i, h, r: (0, i, r, 0)),
            pl.BlockSpec((1, BN, DH), lambda i, h, r: (h, r, 0)),
        ],
        out_specs=[
            pl.BlockSpec((1, 1, BN, DH), lambda i, h, r: (i, h, r, 0)),
            pl.BlockSpec((1, BN, 1), lambda i, h, r: (i, r, 0)),
        ],
        out_shape=[
            jax.ShapeDtypeStruct((NCONV, 2, N, DH), jnp.float32),
            jax.ShapeDtypeStruct((NCONV, N, 1), jnp.float32),
        ],
    )(agg1, deg4, xh[0])

    (agg2,) = _make_edge_pass(NCONV)(tx1h, src, dst, ew)

    out = pl.pallas_call(
        _tc_b_body,
        grid=(N // BN,),
        in_specs=[
            pl.BlockSpec((BN, D), lambda r: (r, 0)),
            pl.BlockSpec((NCONV, 2, BN, DH), lambda r: (0, 0, r, 0)),
            pl.BlockSpec((NCORES, NCONV, 2, BN, DH),
                         lambda r: (0, 0, 0, r, 0)),
            pl.BlockSpec((NCONV, BN, 1), lambda r: (0, r, 0)),
            pl.BlockSpec((NCONV, KCH, D, D), lambda r: (0, 0, 0, 0)),
            pl.BlockSpec((NCONV, D), lambda r: (0, 0)),
        ],
        out_specs=pl.BlockSpec((BN, D), lambda r: (r, 0)),
        out_shape=jax.ShapeDtypeStruct((N, D), jnp.float32),
    )(x, tx1h, agg2, wdiag, W_, b)
    return out


# speedup vs baseline: 5.8965x; 2.3729x over previous
import functools

import jax
import jax.numpy as jnp
from jax import lax
from jax.experimental import pallas as pl
from jax.experimental.pallas import tpu as pltpu
from jax.experimental.pallas import tpu_sc as plsc

NCONV = 3
KCH = 3
N = 10000
E = 320000
D = 128
DH = D // 2

NCORES = 2
NSUB = 16
NW = NCORES * NSUB
EW = E // NW
C = 64
EWP = 10240
NCH = EWP // C
W = 80
NWIN = NCH // W
NB = 4
RT = N // NSUB
RTA = 640
RTL = N - 15 * RTA
ZC = 40
DV = DH // 16


def _sc_mesh():
    return plsc.VectorSubcoreMesh(core_axis_name="c", subcore_axis_name="s",
                                  num_cores=NCORES, num_subcores=NSUB)


def _deg_body(src_hbm, ew_hbm, rid_hbm, deg_out,
              acc_deg, src_v, ew_v, rid_v, deg_l, deg_l2):
    cid = lax.axis_index("c")
    sid = lax.axis_index("s")
    wid = cid * NSUB + sid
    zeros16 = jnp.zeros((16,), jnp.float32)

    pltpu.sync_copy(rid_hbm, rid_v)

    @pl.loop(0, NCONV)
    def _(i):
        @pl.loop(0, RT)
        def _(j):
            deg_l[pl.ds(j * 16, 16)] = zeros16
            deg_l2[j] = zeros16

        @pl.when(sid == 0)
        def _():
            pltpu.sync_copy(deg_l2, acc_deg)

        pltpu.sync_copy(src_hbm.at[i, wid], src_v)
        pltpu.sync_copy(ew_hbm.at[i, wid], ew_v)

        plsc.subcore_barrier()

        @pl.loop(0, NCH)
        def _(j):
            for k in range(C // 16):
                sl = pl.ds(k * 16, 16)
                plsc.addupdate_scatter(deg_l, [src_v[j, sl]], ew_v[j, sl])

        @pl.loop(0, RT)
        def _(j):
            deg_l2[j] = deg_l[pl.ds(j * 16, 16)]

        for k in range(5):
            pltpu.sync_copy(deg_l2.at[pl.ds(k * 125, 125)],
                            acc_deg.at[rid_v.at[k]], add=True)

        plsc.subcore_barrier()

        @pl.when(sid == 0)
        def _():
            pltpu.sync_copy(acc_deg, deg_out.at[cid, i])


@functools.lru_cache(maxsize=None)
def _make_deg_pass():
    return pl.kernel(
        _deg_body,
        out_type=(jax.ShapeDtypeStruct((NCORES, NCONV, RT, 16), jnp.float32),),
        mesh=_sc_mesh(),
        scratch_types=[
            pltpu.VMEM_SHARED((RT, 16), jnp.float32),
            pltpu.VMEM((NCH, C), jnp.int32),
            pltpu.VMEM((NCH, C), jnp.float32),
            pltpu.VMEM((5, 125), jnp.int32),
            pltpu.VMEM((N,), jnp.float32),
            pltpu.VMEM((RT, 16), jnp.float32),
        ],
        compiler_params=pltpu.CompilerParams(
            needs_layout_passes=False, use_tc_tiling_on_sc=False),
    )


def _edge_body(nconv_tables, tables, src_hbm, dst_hbm, ew_hbm, agg_out,
               xs, acc, src_w, dst_w, ew_w, rows, zbuf, gsem, ssem):
    cid = lax.axis_index("c")
    sid = lax.axis_index("s")
    wid = cid * NSUB + sid

    _zero = jnp.zeros((32,), jnp.bfloat16)

    @pl.loop(0, ZC)
    def _(j):
        for v in range(DH // 32):
            zbuf[j, pl.ds(v * 32, 32)] = _zero

    @pl.loop(0, 2)
    def _(h):
        @pl.loop(0, NCONV)
        def _(i):
            ti = i if nconv_tables > 1 else 0

            @pl.when(sid < 15)
            def _():
                pltpu.sync_copy(tables.at[ti, h, pl.ds(sid * RTA, RTA)],
                                xs.at[pl.ds(sid * RTA, RTA)])

                @pl.loop(0, RTA // ZC)
                def _(z):
                    pltpu.sync_copy(zbuf,
                                    acc.at[pl.ds(sid * RTA + z * ZC, ZC)])

            @pl.when(sid == 15)
            def _():
                pltpu.sync_copy(tables.at[ti, h, pl.ds(15 * RTA, RTL)],
                                xs.at[pl.ds(15 * RTA, RTL)])

                @pl.loop(0, RTL // ZC)
                def _(z):
                    pltpu.sync_copy(zbuf,
                                    acc.at[pl.ds(15 * RTA + z * ZC, ZC)])

            plsc.subcore_barrier()

            @pl.loop(0, NWIN)
            def _(w):
                pltpu.sync_copy(src_hbm.at[i, wid, pl.ds(w * W, W)], src_w)
                pltpu.sync_copy(dst_hbm.at[i, wid, pl.ds(w * W, W)], dst_w)
                pltpu.sync_copy(ew_hbm.at[i, wid, pl.ds(w * W, W)], ew_w)

                @pl.loop(0, W, step=NB)
                def _(j0):
                    gds = []
                    for bb in range(NB):
                        gds.append(pltpu.async_copy(
                            xs.at[src_w.at[j0 + bb]],
                            rows.at[pl.ds(bb * C, C)], gsem.at[bb]))
                    sds = []
                    for bb in range(NB):
                        j = j0 + bb
                        gds[bb].wait()

                        @pl.loop(0, C // 16)
                        def _(k):
                            wneg = ew_w[j, pl.ds(k * 16, 16)] * -1.0
                            for t in range(16):
                                r = bb * C + k * 16 + t
                                vwf = jnp.broadcast_to(wneg[t], (16,))
                                vw = plsc.pack(
                                    vwf, vwf,
                                    format=plsc.PackFormat.INTERLEAVED)
                                for v in range(DH // 32):
                                    sl = pl.ds(v * 32, 32)
                                    rows[r, sl] = rows[r, sl] * vw
                        sds.append(pltpu.async_copy(
                            rows.at[pl.ds(bb * C, C)],
                            acc.at[dst_w.at[j]], ssem.at[bb], add=True))
                    for dsc in sds:
                        dsc.wait()

            plsc.subcore_barrier()

            @pl.when(sid < 15)
            def _():
                pltpu.sync_copy(acc.at[pl.ds(sid * RTA, RTA)],
                                agg_out.at[cid, i, h, pl.ds(sid * RTA, RTA)])

            @pl.when(sid == 15)
            def _():
                pltpu.sync_copy(acc.at[pl.ds(15 * RTA, RTL)],
                                agg_out.at[cid, i, h, pl.ds(15 * RTA, RTL)])


@functools.lru_cache(maxsize=None)
def _make_edge_pass(nconv_tables):
    body = functools.partial(_edge_body, nconv_tables)

    return pl.kernel(
        body,
        out_type=(jax.ShapeDtypeStruct((NCORES, NCONV, 2, N, DH),
                                       jnp.bfloat16),),
        mesh=_sc_mesh(),
        scratch_types=[
            pltpu.VMEM_SHARED((N, DH), jnp.bfloat16),
            pltpu.VMEM_SHARED((N, DH), jnp.bfloat16),
            pltpu.VMEM((W, C), jnp.int32),
            pltpu.VMEM((W, C), jnp.int32),
            pltpu.VMEM((W, C), jnp.float32),
            pltpu.VMEM((NB * C, DH), jnp.bfloat16),
            pltpu.VMEM((ZC, DH), jnp.bfloat16),
            pltpu.SemaphoreType.DMA((NB,)),
            pltpu.SemaphoreType.DMA((NB,)),
        ],
        compiler_params=pltpu.CompilerParams(
            needs_layout_passes=False, use_tc_tiling_on_sc=False),
    )


BN = 1000


def _tc_a_body(agg_ref, deg_ref, xh_ref, tx1h_ref, tx1hb_ref, wdiag_ref):
    wd = deg_ref[0, 0] + deg_ref[1, 0] - 1.0
    wdiag_ref[0] = wd
    t1 = (agg_ref[0, 0, 0].astype(jnp.float32)
          + agg_ref[1, 0, 0].astype(jnp.float32) + wd * xh_ref[0])
    tx1h_ref[0, 0] = t1
    tx1hb_ref[0, 0] = t1.astype(jnp.bfloat16)


def _tc_b_body(x_ref, tx1h_ref, agg2_ref, wdiag_ref, w_ref, b_ref, out_ref):
    xb = x_ref[...]
    acc = b_ref[0:1] + b_ref[1:2] + b_ref[2:3]
    acc = jnp.broadcast_to(acc, (BN, D)).astype(jnp.float32)
    for i in range(NCONV):
        t1 = jnp.concatenate([tx1h_ref[i, 0], tx1h_ref[i, 1]], axis=-1)
        a2 = jnp.concatenate(
            [agg2_ref[0, i, 0].astype(jnp.float32)
             + agg2_ref[1, i, 0].astype(jnp.float32),
             agg2_ref[0, i, 1].astype(jnp.float32)
             + agg2_ref[1, i, 1].astype(jnp.float32)], axis=-1)
        t2 = 2.0 * (a2 + wdiag_ref[i] * t1) - xb
        acc = acc + jnp.dot(xb, w_ref[i, 0], precision=lax.Precision.HIGHEST,
                            preferred_element_type=jnp.float32)
        acc = acc + jnp.dot(t1, w_ref[i, 1], precision=lax.Precision.HIGHEST,
                            preferred_element_type=jnp.float32)
        acc = acc + jnp.dot(t2, w_ref[i, 2], precision=lax.Precision.HIGHEST,
                            preferred_element_type=jnp.float32)
    out_ref[...] = acc


def _pad_edges(a, fill):
    pad = jnp.full((NCONV, NW, EWP - EW), fill, a.dtype)
    return jnp.concatenate([a.reshape(NCONV, NW, EW), pad], axis=2).reshape(
        NCONV, NW, NCH, C)


@jax.jit
def kernel(x, edge_index_list, edge_weights_list, W_, b):
    src = _pad_edges(edge_index_list[:, 0], 0)
    dst = _pad_edges(edge_index_list[:, 1], 0)
    ew = _pad_edges(edge_weights_list, 0.0)
    rid = jnp.arange(RT, dtype=jnp.int32).reshape(5, 125)
    xh = jnp.stack([x[:, :DH], x[:, DH:]])[None].astype(
        jnp.bfloat16)

    (deg,) = _make_deg_pass()(src, ew, rid)
    deg4 = deg.reshape(NCORES, NCONV, N, 1)

    (agg1,) = _make_edge_pass(1)(xh, src, dst, ew)

    tx1h, tx1hb, wdiag = pl.pallas_call(
        _tc_a_body,
        grid=(NCONV, 2, N // BN),
        in_specs=[
            pl.BlockSpec((NCORES, 1, 1, BN, DH),
                         lambda i, h, r: (0, i, h, r, 0)),
            pl.BlockSpec((NCORES, 1, BN, 1), lambda i, h, r: (0, i, r, 0)),
            pl.BlockSpec((1, BN, DH), lambda i, h, r: (h, r, 0)),
        ],
        out_specs=[
            pl.BlockSpec((1, 1, BN, DH), lambda i, h, r: (i, h, r, 0)),
            pl.BlockSpec((1, 1, BN, DH), lambda i, h, r: (i, h, r, 0)),
            pl.BlockSpec((1, BN, 1), lambda i, h, r: (i, r, 0)),
        ],
        out_shape=[
            jax.ShapeDtypeStruct((NCONV, 2, N, DH), jnp.float32),
            jax.ShapeDtypeStruct((NCONV, 2, N, DH), jnp.bfloat16),
            jax.ShapeDtypeStruct((NCONV, N, 1), jnp.float32),
        ],
    )(agg1, deg4, x.reshape(N, 2, DH).transpose(1, 0, 2))

    (agg2,) = _make_edge_pass(NCONV)(tx1hb, src, dst, ew)

    out = pl.pallas_call(
        _tc_b_body,
        grid=(N // BN,),
        in_specs=[
            pl.BlockSpec((BN, D), lambda r: (r, 0)),
            pl.BlockSpec((NCONV, 2, BN, DH), lambda r: (0, 0, r, 0)),
            pl.BlockSpec((NCORES, NCONV, 2, BN, DH),
                         lambda r: (0, 0, 0, r, 0)),
            pl.BlockSpec((NCONV, BN, 1), lambda r: (0, r, 0)),
            pl.BlockSpec((NCONV, KCH, D, D), lambda r: (0, 0, 0, 0)),
            pl.BlockSpec((NCONV, D), lambda r: (0, 0)),
        ],
        out_specs=pl.BlockSpec((BN, D), lambda r: (r, 0)),
        out_shape=jax.ShapeDtypeStruct((N, D), jnp.float32),
    )(x, tx1h, agg2, wdiag, W_, b)
    return out
